# Initial kernel scaffold; baseline (speedup 1.0000x reference)
import jax
import jax.numpy as jnp
from jax.experimental import pallas as pl


def kernel(x, edge_index, edge_attr, batch_idx, W0, b0, W_rel, W_root, b_rgcn, W_mf_l, b_mf, W_mf_r, W1, b1, W2, b2):
    raise NotImplementedError("write your pallas kernel here")


# speedup vs baseline: 3.2086x; 3.2086x over previous
import functools

import jax
import jax.numpy as jnp
from jax import lax
from jax.experimental import pallas as pl
from jax.experimental.pallas import tpu as pltpu
from jax.experimental.pallas import tpu_sc as plsc

N = 10000
E = 320000
D = 128
R = 16
MAXD = 10
NG = 64

NC, NS, L = 2, 16, 16
NW = NC * NS
EW = E // NW
CB = 80
NCH = EW // CB

TILE = 1000
NT = N // TILE

f32 = jnp.float32
i32 = jnp.int32

_MESH = plsc.VectorSubcoreMesh(
    core_axis_name="c", subcore_axis_name="s", num_cores=NC, num_subcores=NS)


def _sc_counts(dst, et, z128):
  @functools.partial(
      pl.kernel,
      out_type=jax.ShapeDtypeStruct((NC, N, D), f32),
      mesh=_MESH,
      scratch_types=[
          pltpu.VMEM_SHARED((N, D), f32),
          pltpu.VMEM((CB,), i32),
          pltpu.VMEM((CB,), i32),
          pltpu.VMEM((CB, D), f32),
      ],
  )
  def k(dst_hbm, et_hbm, z_hbm, out_hbm, acc, dstv, etv, oh):
    cid = lax.axis_index("c")
    sid = lax.axis_index("s")
    wid = sid * NC + cid

    @pl.when(sid == 0)
    def _():
      pltpu.sync_copy(z_hbm, acc)

    iot = lax.broadcasted_iota(i32, (L,), 0)
    one = jnp.ones((L,), f32)
    zero = jnp.zeros((L,), f32)
    for q in range(CB):
      for cg in range(D // L):
        oh[q, pl.ds(cg * L, L)] = zero
    plsc.subcore_barrier()

    def body(c, carry):
      base = wid * EW + c * CB
      pltpu.sync_copy(dst_hbm.at[pl.ds(base, CB)], dstv)
      pltpu.sync_copy(et_hbm.at[pl.ds(base, CB)], etv)
      for g in range(CB // L):
        e16 = etv[pl.ds(g * L, L)]
        for kk in range(L):
          oh[g * L + kk, pl.ds(0, L)] = jnp.where(iot == e16[kk], one, zero)
      pltpu.sync_copy(oh, acc.at[dstv], add=True)
      return carry

    lax.fori_loop(0, NCH, body, 0)
    plsc.subcore_barrier()

    @pl.when(sid == 0)
    def _():
      pltpu.sync_copy(acc, out_hbm.at[cid])

  return k(dst, et, z128)


def _sc_edgew(dst, et, normA, normB):
  @functools.partial(
      pl.kernel,
      out_type=jax.ShapeDtypeStruct((E,), f32),
      mesh=_MESH,
      compiler_params=pltpu.CompilerParams(needs_layout_passes=False),
      scratch_types=[
          pltpu.VMEM((N * 8,), f32),
          pltpu.VMEM((CB,), i32),
          pltpu.VMEM((CB,), i32),
          pltpu.VMEM((EW,), f32),
      ],
  )
  def k(dst_hbm, et_hbm, na_hbm, nb_hbm, w_hbm, tab, dstv, etv, wv):
    cid = lax.axis_index("c")
    sid = lax.axis_index("s")
    wid = sid * NC + cid
    zero = jnp.zeros((L,), f32)

    pltpu.sync_copy(na_hbm, tab)

    def bodyA(c, carry):
      base = wid * EW + c * CB
      pltpu.sync_copy(dst_hbm.at[pl.ds(base, CB)], dstv)
      pltpu.sync_copy(et_hbm.at[pl.ds(base, CB)], etv)
      for g in range(CB // L):
        sl = pl.ds(g * L, L)
        d16 = dstv[sl]
        e16 = etv[sl]
        lo = e16 < 8
        idx = d16 * 8 + jnp.where(lo, e16, 0)
        g16 = plsc.load_gather(tab, [idx])
        wv[pl.ds(c * CB + g * L, L)] = jnp.where(lo, g16, zero)
      return carry

    lax.fori_loop(0, NCH, bodyA, 0)
    pltpu.sync_copy(nb_hbm, tab)

    def bodyB(c, carry):
      base = wid * EW + c * CB
      pltpu.sync_copy(dst_hbm.at[pl.ds(base, CB)], dstv)
      pltpu.sync_copy(et_hbm.at[pl.ds(base, CB)], etv)
      for g in range(CB // L):
        sl = pl.ds(g * L, L)
        d16 = dstv[sl]
        e16 = etv[sl]
        hi = e16 >= 8
        idx = d16 * 8 + jnp.where(hi, e16 - 8, 0)
        g16 = plsc.load_gather(tab, [idx])
        wsl = pl.ds(c * CB + g * L, L)
        wv[wsl] = wv[wsl] + jnp.where(hi, g16, zero)
      return carry

    lax.fori_loop(0, NCH, bodyB, 0)
    pltpu.sync_copy(wv, w_hbm.at[pl.ds(wid * EW, EW)])

  return k(dst, et, normA, normB)


def _sc_rgcn(xw, w, src, dst, et, z128):
  @functools.partial(
      pl.kernel,
      out_type=jax.ShapeDtypeStruct((NC, N, D), f32),
      mesh=_MESH,
      scratch_types=[
          pltpu.VMEM_SHARED((N, D), f32),
          pltpu.VMEM((CB,), i32),
          pltpu.VMEM((CB,), i32),
          pltpu.VMEM((CB,), i32),
          pltpu.VMEM((CB,), i32),
          pltpu.VMEM((CB,), f32),
          pltpu.VMEM((CB, D), f32),
          pltpu.SemaphoreType.DMA,
      ],
  )
  def k(xw_hbm, w_hbm, src_hbm, dst_hbm, et_hbm, z_hbm, out_hbm,
        acc, srcv, etv, dstv, gidx, wchunk, rows, sem1):
    cid = lax.axis_index("c")
    sid = lax.axis_index("s")
    wid = sid * NC + cid

    @pl.when(sid == 0)
    def _():
      pltpu.sync_copy(z_hbm, acc)

    plsc.subcore_barrier()

    def body(c, carry):
      base = wid * EW + c * CB
      pltpu.sync_copy(src_hbm.at[pl.ds(base, CB)], srcv)
      pltpu.sync_copy(et_hbm.at[pl.ds(base, CB)], etv)
      pltpu.sync_copy(dst_hbm.at[pl.ds(base, CB)], dstv)
      pltpu.sync_copy(w_hbm.at[pl.ds(base, CB)], wchunk)
      for g in range(CB // L):
        sl = pl.ds(g * L, L)
        gidx[sl] = etv[sl] * N + srcv[sl]
      pltpu.async_copy(xw_hbm.at[gidx], rows, sem1).wait()
      for g in range(CB // L):
        w16 = wchunk[pl.ds(g * L, L)]
        for kk in range(L):
          w = w16[kk]
          j = g * L + kk
          for cg in range(D // L):
            sl = pl.ds(cg * L, L)
            rows[j, sl] = rows[j, sl] * w
      pltpu.sync_copy(rows, acc.at[dstv], add=True)
      return carry

    lax.fori_loop(0, NCH, body, 0)
    plsc.subcore_barrier()

    @pl.when(sid == 0)
    def _():
      pltpu.sync_copy(acc, out_hbm.at[cid])

  return k(xw, w, src, dst, et, z128)


def _sc_nbrsum(h, src, dst, z128):
  @functools.partial(
      pl.kernel,
      out_type=jax.ShapeDtypeStruct((NC, N, D), f32),
      mesh=_MESH,
      scratch_types=[
          pltpu.VMEM_SHARED((N, D), f32),
          pltpu.VMEM((CB,), i32),
          pltpu.VMEM((CB,), i32),
          pltpu.VMEM((CB, D), f32),
          pltpu.SemaphoreType.DMA,
      ],
  )
  def k(h_hbm, src_hbm, dst_hbm, z_hbm, out_hbm,
        acc, srcv, dstv, rows, sem1):
    cid = lax.axis_index("c")
    sid = lax.axis_index("s")
    wid = sid * NC + cid

    @pl.when(sid == 0)
    def _():
      pltpu.sync_copy(z_hbm, acc)

    plsc.subcore_barrier()

    def body(c, carry):
      base = wid * EW + c * CB
      pltpu.sync_copy(src_hbm.at[pl.ds(base, CB)], srcv)
      pltpu.sync_copy(dst_hbm.at[pl.ds(base, CB)], dstv)
      pltpu.async_copy(h_hbm.at[srcv], rows, sem1).wait()
      pltpu.sync_copy(rows, acc.at[dstv], add=True)
      return carry

    lax.fori_loop(0, NCH, body, 0)
    plsc.subcore_barrier()

    @pl.when(sid == 0)
    def _():
      pltpu.sync_copy(acc, out_hbm.at[cid])

  return k(h, src, dst, z128)


def _tc_linear(x, W, b):
  def body(x_ref, w_ref, b_ref, o_ref):
    o_ref[...] = (jnp.dot(x_ref[...], w_ref[...],
                          preferred_element_type=f32) + b_ref[...])

  return pl.pallas_call(
      body,
      grid=(NT,),
      in_specs=[
          pl.BlockSpec((TILE, D), lambda t: (t, 0)),
          pl.BlockSpec((D, D), lambda t: (0, 0)),
          pl.BlockSpec((1, D), lambda t: (0, 0)),
      ],
      out_specs=pl.BlockSpec((TILE, D), lambda t: (t, 0)),
      out_shape=jax.ShapeDtypeStruct((N, D), f32),
  )(x, W, b.reshape(1, D))


def _tc_xw(h, Wrel):
  def body(h_ref, w_ref, o_ref):
    hr = jnp.maximum(h_ref[...], 0.0)
    o_ref[...] = jnp.dot(hr, w_ref[0], preferred_element_type=f32)

  return pl.pallas_call(
      body,
      grid=(R, NT),
      in_specs=[
          pl.BlockSpec((TILE, D), lambda r, t: (t, 0)),
          pl.BlockSpec((1, D, D), lambda r, t: (r, 0, 0)),
      ],
      out_specs=pl.BlockSpec((TILE, D), lambda r, t: (r * NT + t, 0)),
      out_shape=jax.ShapeDtypeStruct((R * N, D), f32),
  )(h, Wrel)


def _tc_norm(cnt2):
  def body(c_ref, n_ref, d_ref):
    cnt = (c_ref[0] + c_ref[1])[:, :R]
    n_ref[...] = 1.0 / jnp.maximum(cnt, 1.0)
    d_ref[...] = jnp.minimum(jnp.sum(cnt, axis=1, keepdims=True), float(MAXD))

  return pl.pallas_call(
      body,
      out_shape=(jax.ShapeDtypeStruct((N, R), f32),
                 jax.ShapeDtypeStruct((N, 1), f32)),
  )(cnt2)


def _tc_rgcn_combine(agg2, h, Wroot, b):
  def body(a_ref, h_ref, w_ref, b_ref, o_ref):
    hr = jnp.maximum(h_ref[...], 0.0)
    v = (a_ref[0] + a_ref[1]
         + jnp.dot(hr, w_ref[...], preferred_element_type=f32) + b_ref[...])
    o_ref[...] = jnp.maximum(v, 0.0)

  return pl.pallas_call(
      body,
      grid=(NT,),
      in_specs=[
          pl.BlockSpec((NC, TILE, D), lambda t: (0, t, 0)),
          pl.BlockSpec((TILE, D), lambda t: (t, 0)),
          pl.BlockSpec((D, D), lambda t: (0, 0)),
          pl.BlockSpec((1, D), lambda t: (0, 0)),
      ],
      out_specs=pl.BlockSpec((TILE, D), lambda t: (t, 0)),
      out_shape=jax.ShapeDtypeStruct((N, D), f32),
  )(agg2, h, Wroot, b.reshape(1, D))


def _tc_mfconv(hs2, hm, degf, Wl, bl, Wr):
  def body(hs_ref, hm_ref, dg_ref, wl_ref, bl_ref, wr_ref, o_ref):
    hs = hs_ref[0] + hs_ref[1]
    x = hm_ref[...]
    dg = dg_ref[...]
    acc = jnp.zeros((TILE, D), f32)
    for d in range(MAXD + 1):
      m = (dg == float(d)).astype(f32)
      acc = acc + m * (jnp.dot(hs, wl_ref[d], preferred_element_type=f32)
                       + bl_ref[d]
                       + jnp.dot(x, wr_ref[d], preferred_element_type=f32))
    o_ref[...] = acc

  return pl.pallas_call(
      body,
      grid=(NT,),
      in_specs=[
          pl.BlockSpec((NC, TILE, D), lambda t: (0, t, 0)),
          pl.BlockSpec((TILE, D), lambda t: (t, 0)),
          pl.BlockSpec((TILE, 1), lambda t: (t, 0)),
          pl.BlockSpec((MAXD + 1, D, D), lambda t: (0, 0, 0)),
          pl.BlockSpec((MAXD + 1, D), lambda t: (0, 0)),
          pl.BlockSpec((MAXD + 1, D, D), lambda t: (0, 0, 0)),
      ],
      out_specs=pl.BlockSpec((TILE, D), lambda t: (t, 0)),
      out_shape=jax.ShapeDtypeStruct((N, D), f32),
  )(hs2, hm, degf, Wl, bl, Wr)


def _tc_pool_mlp(h, batchi, W1, b1, W2, b2):
  def body(h_ref, bi_ref, w1_ref, b1_ref, w2_ref, b2_ref, o_ref, acc_ref):
    t = pl.program_id(0)

    @pl.when(t == 0)
    def _():
      acc_ref[...] = jnp.zeros((NG, D), f32)

    gi = lax.broadcasted_iota(i32, (NG, TILE), 0)
    oh = (gi == bi_ref[0]).astype(f32)
    acc_ref[...] += jnp.dot(oh, h_ref[...], preferred_element_type=f32)

    @pl.when(t == NT - 1)
    def _():
      z = jnp.maximum(
          jnp.dot(acc_ref[...], w1_ref[...], preferred_element_type=f32)
          + b1_ref[...], 0.0)
      o_ref[...] = (jnp.dot(z, w2_ref[...], preferred_element_type=f32)
                    + b2_ref[...])

  return pl.pallas_call(
      body,
      grid=(NT,),
      in_specs=[
          pl.BlockSpec((TILE, D), lambda t: (t, 0)),
          pl.BlockSpec((1, 1, TILE), lambda t: (t, 0, 0)),
          pl.BlockSpec((D, D), lambda t: (0, 0)),
          pl.BlockSpec((1, D), lambda t: (0, 0)),
          pl.BlockSpec((D, D), lambda t: (0, 0)),
          pl.BlockSpec((1, D), lambda t: (0, 0)),
      ],
      out_specs=pl.BlockSpec((NG, D), lambda t: (0, 0)),
      out_shape=jax.ShapeDtypeStruct((NG, D), f32),
      scratch_shapes=[pltpu.VMEM((NG, D), f32)],
  )(h, batchi, W1, b1.reshape(1, D), W2, b2.reshape(1, D))


def kernel(x, edge_index, edge_attr, batch_idx,
           W0, b0, W_rel, W_root, b_rgcn, W_mf_l, b_mf, W_mf_r,
           W1, b1, W2, b2):
  src = edge_index[0].astype(i32)
  dst = edge_index[1].astype(i32)
  et = edge_attr.astype(i32)
  z128 = jnp.zeros((N, D), f32)

  cnt2 = _sc_counts(dst, et, z128)
  normtab, degf = _tc_norm(cnt2)
  normA = normtab[:, :8].reshape(N * 8)
  normB = normtab[:, 8:].reshape(N * 8)
  ew = _sc_edgew(dst, et, normA, normB)

  h = _tc_linear(x, W0, b0)
  for blk in range(2):
    xw = _tc_xw(h, W_rel[blk])
    agg2 = _sc_rgcn(xw, ew, src, dst, et, z128)
    hm = _tc_rgcn_combine(agg2, h, W_root[blk], b_rgcn[blk])
    hs2 = _sc_nbrsum(hm, src, dst, z128)
    h = _tc_mfconv(hs2, hm, degf, W_mf_l[blk], b_mf[blk], W_mf_r[blk])

  return _tc_pool_mlp(h, batch_idx.astype(i32).reshape(NT, 1, TILE),
                      W1, b1, W2, b2)
